# baseline (speedup 1.0000x reference)
import functools

import jax
import jax.numpy as jnp
from jax import lax
from jax.experimental import pallas as pl
from jax.experimental.pallas import tpu as pltpu
from jax.experimental.pallas import tpu_sc as plsc

_NUM_BUCKETS = 1000
_EMBED_DIM = 128
_BATCH = 16384
_DIV = 100.0 + 1e-8

_NC = 2
_NS = 16
_L = 16
_NW = _NC * _NS
_BPW = _BATCH // _NW
_CHUNK = 128
_NCHUNK = _BPW // _CHUNK


def _body(x_hbm, table_hbm, out_hbm, x_v, idx_v, rows_v, gsem, ssem):
    wid = lax.axis_index("s") * _NC + lax.axis_index("c")
    base = wid * _BPW
    pltpu.sync_copy(x_hbm.at[pl.ds(base, _BPW)], x_v)

    gathers = []
    for j in range(_NCHUNK):
        for i in range(_CHUNK // _L):
            xv = x_v[pl.ds(j * _CHUNK + i * _L, _L)]
            y = (xv / jnp.float32(_DIV)) * jnp.float32(_NUM_BUCKETS)
            idx = jnp.clip(y.astype(jnp.int32), 0, _NUM_BUCKETS - 1)
            idx_v[j, pl.ds(i * _L, _L)] = idx
        gathers.append(
            pltpu.async_copy(
                table_hbm.at[idx_v.at[j]],
                rows_v.at[pl.ds(j * _CHUNK, _CHUNK)],
                gsem,
            )
        )
    stores = []
    for j in range(_NCHUNK):
        gathers[j].wait()
        stores.append(
            pltpu.async_copy(
                rows_v.at[pl.ds(j * _CHUNK, _CHUNK)],
                out_hbm.at[pl.ds(base + j * _CHUNK, _CHUNK)],
                ssem,
            )
        )
    for s in stores:
        s.wait()


_sc_lookup = functools.partial(
    pl.kernel,
    out_type=jax.ShapeDtypeStruct((_BATCH, _EMBED_DIM), jnp.float32),
    mesh=plsc.VectorSubcoreMesh(core_axis_name="c", subcore_axis_name="s"),
    scratch_types=[
        pltpu.VMEM((_BPW,), jnp.float32),
        pltpu.VMEM((_NCHUNK, _CHUNK), jnp.int32),
        pltpu.VMEM((_BPW, _EMBED_DIM), jnp.float32),
        pltpu.SemaphoreType.DMA,
        pltpu.SemaphoreType.DMA,
    ],
)(_body)


def kernel(x, table):
    return _sc_lookup(x, table)


# speedup vs baseline: 1.1544x; 1.1544x over previous
import functools

import jax
import jax.numpy as jnp
from jax import lax
from jax.experimental import pallas as pl
from jax.experimental.pallas import tpu as pltpu
from jax.experimental.pallas import tpu_sc as plsc

_NUM_BUCKETS = 1000
_EMBED_DIM = 128
_BATCH = 16384
_DIV = 100.0 + 1e-8

_NC = 2
_NS = 16
_L = 16
_NW = _NC * _NS
_BPW = _BATCH // _NW
_CHUNK = 128
_NCHUNK = _BPW // _CHUNK
_ROWS = _NUM_BUCKETS
_RPT = 64


def _body(x_hbm, table_hbm, out_hbm, x_v, idx_v, rows_v, table_sh, gsems, ssem):
    cid = lax.axis_index("c")
    sid = lax.axis_index("s")
    wid = sid * _NC + cid
    base = wid * _BPW

    start = pl.multiple_of(jnp.minimum(sid * _RPT, _ROWS - _RPT), 8)
    pltpu.sync_copy(
        table_hbm.at[pl.ds(start, _RPT)], table_sh.at[pl.ds(start, _RPT)]
    )

    pltpu.sync_copy(x_hbm.at[pl.ds(base, _BPW)], x_v)
    for j in range(_NCHUNK):
        for i in range(_CHUNK // _L):
            xv = x_v[pl.ds(j * _CHUNK + i * _L, _L)]
            y = (xv / jnp.float32(_DIV)) * jnp.float32(_NUM_BUCKETS)
            idx = jnp.clip(y.astype(jnp.int32), 0, _NUM_BUCKETS - 1)
            idx_v[j, pl.ds(i * _L, _L)] = idx

    plsc.subcore_barrier()

    gathers = [
        pltpu.async_copy(
            table_sh.at[idx_v.at[j]],
            rows_v.at[pl.ds(j * _CHUNK, _CHUNK)],
            gsems[j],
        )
        for j in range(_NCHUNK)
    ]
    for g in gathers:
        g.wait()
    pltpu.sync_copy(rows_v, out_hbm.at[pl.ds(base, _BPW)])


_sc_lookup = functools.partial(
    pl.kernel,
    out_type=jax.ShapeDtypeStruct((_BATCH, _EMBED_DIM), jnp.float32),
    mesh=plsc.VectorSubcoreMesh(core_axis_name="c", subcore_axis_name="s"),
    scratch_types=[
        pltpu.VMEM((_BPW,), jnp.float32),
        pltpu.VMEM((_NCHUNK, _CHUNK), jnp.int32),
        pltpu.VMEM((_BPW, _EMBED_DIM), jnp.float32),
        pltpu.VMEM_SHARED((_ROWS, _EMBED_DIM), jnp.float32),
        [pltpu.SemaphoreType.DMA] * _NCHUNK,
        pltpu.SemaphoreType.DMA,
    ],
)(_body)


def kernel(x, table):
    return _sc_lookup(x, table)


# speedup vs baseline: 1.1873x; 1.0285x over previous
import functools

import jax
import jax.numpy as jnp
from jax import lax
from jax.experimental import pallas as pl
from jax.experimental.pallas import tpu as pltpu
from jax.experimental.pallas import tpu_sc as plsc

_NUM_BUCKETS = 1000
_EMBED_DIM = 128
_BATCH = 16384
_DIV = 100.0 + 1e-8

_NC = 2
_NS = 16
_L = 16
_NW = _NC * _NS
_BPW = _BATCH // _NW
_CHUNK = 128
_NCHUNK = _BPW // _CHUNK
_ROWS = _NUM_BUCKETS
_RPT = 64


def _body(x_hbm, table_hbm, out_hbm, x_v, idx_v, rows_v, table_sh, gsems, ssem):
    cid = lax.axis_index("c")
    sid = lax.axis_index("s")
    wid = sid * _NC + cid
    base = wid * _BPW

    start = pl.multiple_of(jnp.minimum(sid * _RPT, _ROWS - _RPT), 8)
    pltpu.sync_copy(
        table_hbm.at[pl.ds(start, _RPT)], table_sh.at[pl.ds(start, _RPT)]
    )

    pltpu.sync_copy(x_hbm.at[pl.ds(base, _BPW)], x_v)
    for j in range(_NCHUNK):
        for i in range(_CHUNK // _L):
            xv = x_v[pl.ds(j * _CHUNK + i * _L, _L)]
            y = (xv / jnp.float32(_DIV)) * jnp.float32(_NUM_BUCKETS)
            idx = jnp.clip(y.astype(jnp.int32), 0, _NUM_BUCKETS - 1)
            idx_v[j, pl.ds(i * _L, _L)] = idx

    plsc.subcore_barrier()

    gathers = [
        pltpu.async_copy(
            table_sh.at[idx_v.at[j]],
            rows_v.at[pl.ds(j * _CHUNK, _CHUNK)],
            gsems[j],
        )
        for j in range(_NCHUNK)
    ]
    for j in range(_NCHUNK):
        gathers[j].wait()
        pltpu.sync_copy(
            rows_v.at[pl.ds(j * _CHUNK, _CHUNK)],
            out_hbm.at[pl.ds(base + j * _CHUNK, _CHUNK)],
        )


_sc_lookup = functools.partial(
    pl.kernel,
    out_type=jax.ShapeDtypeStruct((_BATCH, _EMBED_DIM), jnp.float32),
    mesh=plsc.VectorSubcoreMesh(core_axis_name="c", subcore_axis_name="s"),
    scratch_types=[
        pltpu.VMEM((_BPW,), jnp.float32),
        pltpu.VMEM((_NCHUNK, _CHUNK), jnp.int32),
        pltpu.VMEM((_BPW, _EMBED_DIM), jnp.float32),
        pltpu.VMEM_SHARED((_ROWS, _EMBED_DIM), jnp.float32),
        [pltpu.SemaphoreType.DMA] * _NCHUNK,
        pltpu.SemaphoreType.DMA,
    ],
)(_body)


def kernel(x, table):
    return _sc_lookup(x, table)


# speedup vs baseline: 1.2031x; 1.0134x over previous
import functools

import jax
import jax.numpy as jnp
from jax import lax
from jax.experimental import pallas as pl
from jax.experimental.pallas import tpu as pltpu
from jax.experimental.pallas import tpu_sc as plsc

_NUM_BUCKETS = 1000
_EMBED_DIM = 128
_BATCH = 16384
_DIV = 100.0 + 1e-8

_NC = 2
_NS = 16
_L = 16
_NW = _NC * _NS
_BPW = _BATCH // _NW
_CHUNK = 64
_NCHUNK = _BPW // _CHUNK
_ROWS = _NUM_BUCKETS
_RPT = 64


def _body(x_hbm, table_hbm, out_hbm, x_v, idx_v, rows_v, table_sh, gsems, ssem):
    cid = lax.axis_index("c")
    sid = lax.axis_index("s")
    wid = sid * _NC + cid
    base = wid * _BPW

    start = pl.multiple_of(jnp.minimum(sid * _RPT, _ROWS - _RPT), 8)
    pltpu.sync_copy(
        table_hbm.at[pl.ds(start, _RPT)], table_sh.at[pl.ds(start, _RPT)]
    )

    pltpu.sync_copy(x_hbm.at[pl.ds(base, _BPW)], x_v)
    for j in range(_NCHUNK):
        for i in range(_CHUNK // _L):
            xv = x_v[pl.ds(j * _CHUNK + i * _L, _L)]
            y = (xv / jnp.float32(_DIV)) * jnp.float32(_NUM_BUCKETS)
            idx = jnp.clip(y.astype(jnp.int32), 0, _NUM_BUCKETS - 1)
            idx_v[j, pl.ds(i * _L, _L)] = idx

    plsc.subcore_barrier()

    gathers = [
        pltpu.async_copy(
            table_sh.at[idx_v.at[j]],
            rows_v.at[pl.ds(j * _CHUNK, _CHUNK)],
            gsems[j],
        )
        for j in range(_NCHUNK)
    ]
    for j in range(_NCHUNK):
        gathers[j].wait()
        pltpu.sync_copy(
            rows_v.at[pl.ds(j * _CHUNK, _CHUNK)],
            out_hbm.at[pl.ds(base + j * _CHUNK, _CHUNK)],
        )


_sc_lookup = functools.partial(
    pl.kernel,
    out_type=jax.ShapeDtypeStruct((_BATCH, _EMBED_DIM), jnp.float32),
    mesh=plsc.VectorSubcoreMesh(core_axis_name="c", subcore_axis_name="s"),
    scratch_types=[
        pltpu.VMEM((_BPW,), jnp.float32),
        pltpu.VMEM((_NCHUNK, _CHUNK), jnp.int32),
        pltpu.VMEM((_BPW, _EMBED_DIM), jnp.float32),
        pltpu.VMEM_SHARED((_ROWS, _EMBED_DIM), jnp.float32),
        [pltpu.SemaphoreType.DMA] * _NCHUNK,
        pltpu.SemaphoreType.DMA,
    ],
)(_body)


def kernel(x, table):
    return _sc_lookup(x, table)
